# baseline (speedup 1.0000x reference)
import functools

import jax
import jax.numpy as jnp
from jax import lax
from jax.experimental import pallas as pl
from jax.experimental.pallas import tpu as pltpu
from jax.experimental.pallas import tpu_sc as plsc

_B, _N, _D, _K = 8, 4096, 256, 128


def _gelu(x):
    return 0.5 * x * (1.0 + lax.erf(x * (2.0 ** -0.5)))


def _dot(a, b):
    return lax.dot_general(a, b, (((1,), (1,)), ((), ())),
                           precision=lax.Precision.HIGHEST,
                           preferred_element_type=jnp.float32)


_BPS = 8


def _encode_body(coords_ref, labels_ref, w1_ref, b1_ref, w2_ref, b2_ref,
                 wa1_ref, ba1_ref, lng_ref, lnb_ref, wa2_ref, ba2_ref,
                 out_ref):
    for b in range(_BPS):
        _encode_one(coords_ref[b], labels_ref[pl.ds(b, 1), :], w1_ref,
                    b1_ref, w2_ref, b2_ref, wa1_ref, ba1_ref, lng_ref,
                    lnb_ref, wa2_ref, ba2_ref, out_ref, b)


def _encode_one(coords, labels, w1_ref, b1_ref, w2_ref, b2_ref,
                wa1_ref, ba1_ref, lng_ref, lnb_ref, wa2_ref, ba2_ref,
                out_ref, b):

    ohT = (labels == lax.broadcasted_iota(jnp.int32, (_K, 1), 0)
           ).astype(jnp.bfloat16)
    cat = jnp.concatenate(
        [coords, jnp.ones((1, _N), jnp.float32)], axis=0)
    cat_hi = cat.astype(jnp.bfloat16)
    cat_lo = (cat - cat_hi.astype(jnp.float32)).astype(jnp.bfloat16)
    dotb = lambda a, b: lax.dot_general(
        a, b, (((1,), (1,)), ((), ())),
        preferred_element_type=jnp.float32)
    stats = dotb(ohT, cat_hi) + dotb(ohT, cat_lo)
    statsT = stats.T

    counts = stats[:, 3:4]
    centers = stats[:, 0:3] / jnp.maximum(counts, 1.0)
    countsT = statsT[3:4, :]
    centersT = statsT[0:3, :] / jnp.maximum(countsT, 1.0)

    vmT = (countsT >= 2.0).astype(jnp.float32)
    nv = jnp.sum(vmT)
    denom = jnp.maximum(nv, 1.0)

    dx = centers[:, 0:1] - centersT[0:1, :]
    dy = centers[:, 1:2] - centersT[1:2, :]
    dz = centers[:, 2:3] - centersT[2:3, :]
    sq = dx * dx + dy * dy + dz * dz
    pos = sq > 0.0
    dist = jnp.where(pos, jnp.sqrt(jnp.where(pos, sq, 1.0)), 0.0)

    mean_d = jnp.sum(dist * vmT, axis=1, keepdims=True) / denom
    min_d = jnp.min(jnp.where(vmT > 0.0, dist, 1e9), axis=1, keepdims=True)
    zc = centers[:, 2:3]
    frac = jnp.sum((zc > centersT[2:3, :]).astype(jnp.float32) * vmT,
                   axis=1, keepdims=True) / denom
    rel = jnp.concatenate([mean_d, min_d, zc, frac], axis=1)

    h = _gelu(_dot(rel, w1_ref[...]) + b1_ref[...])
    enc = _dot(h, w2_ref[...]) + b2_ref[...]
    a = _dot(enc, wa1_ref[...]) + ba1_ref[...]
    mu = jnp.mean(a, axis=1, keepdims=True)
    var = jnp.mean((a - mu) ** 2, axis=1, keepdims=True)
    a = (a - mu) / jnp.sqrt(var + 1e-5) * lng_ref[...] + lnb_ref[...]
    a = _gelu(a)
    agg = _dot(a, wa2_ref[...]) + ba2_ref[...]

    seg_ok = (counts >= 2.0) & (nv >= 2.0)
    out_ref[pl.ds(b * _K, _K), :] = jnp.where(seg_ok, agg, 0.0)


def _encode(coords_t, labels, W1, b1, W2, b2, Wa1, ba1, ln_g, ln_b,
            Wa2, ba2):
    full = lambda shape: pl.BlockSpec(shape, lambda b: (0,) * len(shape))
    return pl.pallas_call(
        _encode_body,
        grid=(_B // _BPS,),
        in_specs=[
            pl.BlockSpec((_BPS, 3, _N), lambda b: (b, 0, 0)),
            pl.BlockSpec((_BPS, _N), lambda b: (b, 0)),
            full((_D // 4, 4)), full((1, _D // 4)),
            full((_D, _D // 4)), full((1, _D)),
            full((_D, _D)), full((1, _D)),
            full((1, _D)), full((1, _D)),
            full((_D, _D)), full((1, _D)),
        ],
        out_specs=pl.BlockSpec((_BPS * _K, _D), lambda b: (b, 0)),
        out_shape=jax.ShapeDtypeStruct((_B * _K, _D), jnp.float32),
    )(coords_t, labels, W1, b1.reshape(1, -1), W2, b2.reshape(1, -1),
      Wa1, ba1.reshape(1, -1), ln_g.reshape(1, -1), ln_b.reshape(1, -1),
      Wa2, ba2.reshape(1, -1))


_NC, _NS = 2, 16
_NW = _NC * _NS
_RW = (_B * _N) // _NW
_CH = 128
_NCH = _RW // _CH
_NBUF = 3


@functools.cache
def _make_gather_rows():
    @functools.partial(
        pl.kernel,
        out_type=jax.ShapeDtypeStruct((_B * _N, _D), jnp.float32),
        mesh=plsc.VectorSubcoreMesh(core_axis_name="c", subcore_axis_name="s"),
        scratch_types=[
            pltpu.VMEM((_RW,), jnp.int32),
            pltpu.VMEM((_NBUF, _CH, _D), jnp.float32),
        ] + [pltpu.SemaphoreType.DMA] * (2 * _NBUF),
    )
    def _gather_rows(table_hbm, idx_hbm, out_hbm, idx_v, rows_v, *sems):
        gsems, wsems = sems[:_NBUF], sems[_NBUF:]
        wid = lax.axis_index("s") * _NC + lax.axis_index("c")
        wpb = _N // _RW
        batch = wid // wpb
        pltpu.sync_copy(idx_hbm.at[batch, pl.ds((wid % wpb) * _RW, _RW)],
                        idx_v)
        off = batch * _K
        for j in range(_RW // 16):
            sl = pl.ds(j * 16, 16)
            idx_v[sl] = idx_v[sl] + off
        gathers = [None] * _NCH
        writes = [None] * _NCH

        def start_gather(g):
            gathers[g] = pltpu.async_copy(
                table_hbm.at[idx_v.at[pl.ds(g * _CH, _CH)]],
                rows_v.at[g % _NBUF], gsems[g % _NBUF])

        for g in range(_NBUF - 1):
            start_gather(g)
        for g in range(_NCH):
            gathers[g].wait()
            writes[g] = pltpu.async_copy(
                rows_v.at[g % _NBUF],
                out_hbm.at[pl.ds(wid * _RW + g * _CH, _CH)],
                wsems[g % _NBUF])
            nxt = g + _NBUF - 1
            if nxt < _NCH:
                if nxt >= _NBUF:
                    writes[nxt - _NBUF].wait()
                start_gather(nxt)
        for g in range(max(0, _NCH - _NBUF), _NCH):
            writes[g].wait()

    return _gather_rows


def kernel(coordinates, features, superpoint_labels, W1, b1, W2, b2,
           Wa1, ba1, ln_g, ln_b, Wa2, ba2):
    coords_t = coordinates.transpose(0, 2, 1)
    labels = superpoint_labels.astype(jnp.int32)

    table = _encode(coords_t, labels, W1, b1, W2, b2, Wa1, ba1,
                    ln_g, ln_b, Wa2, ba2)

    out = _make_gather_rows()(table, labels)
    return out.reshape(_B, _N, _D).astype(features.dtype)


# speedup vs baseline: 1.0461x; 1.0461x over previous
import functools

import jax
import jax.numpy as jnp
from jax import lax
from jax.experimental import pallas as pl
from jax.experimental.pallas import tpu as pltpu
from jax.experimental.pallas import tpu_sc as plsc

_B, _N, _D, _K = 8, 4096, 256, 128


def _gelu(x):
    return 0.5 * x * (1.0 + lax.erf(x * (2.0 ** -0.5)))


def _dotb16(a, b):
    return lax.dot_general(a, b, (((1,), (1,)), ((), ())),
                           preferred_element_type=jnp.float32)


def _dot(a, b):
    a_hi = a.astype(jnp.bfloat16)
    a_lo = (a - a_hi.astype(jnp.float32)).astype(jnp.bfloat16)
    b_hi = b.astype(jnp.bfloat16)
    b_lo = (b - b_hi.astype(jnp.float32)).astype(jnp.bfloat16)
    return (_dotb16(a_hi, b_hi) + _dotb16(a_hi, b_lo)
            + _dotb16(a_lo, b_hi))


_BPS = 8


def _encode_body(coords_ref, labels_ref, w1_ref, b1_ref, w2_ref, b2_ref,
                 wa1_ref, ba1_ref, lng_ref, lnb_ref, wa2_ref, ba2_ref,
                 out_ref):
    for b in range(_BPS):
        _encode_one(coords_ref[b], labels_ref[pl.ds(b, 1), :], w1_ref,
                    b1_ref, w2_ref, b2_ref, wa1_ref, ba1_ref, lng_ref,
                    lnb_ref, wa2_ref, ba2_ref, out_ref, b)


def _encode_one(coords, labels, w1_ref, b1_ref, w2_ref, b2_ref,
                wa1_ref, ba1_ref, lng_ref, lnb_ref, wa2_ref, ba2_ref,
                out_ref, b):

    ohT = (labels == lax.broadcasted_iota(jnp.int32, (_K, 1), 0)
           ).astype(jnp.bfloat16)
    cat = jnp.concatenate(
        [coords, jnp.ones((1, _N), jnp.float32)], axis=0)
    cat_hi = cat.astype(jnp.bfloat16)
    cat_lo = (cat - cat_hi.astype(jnp.float32)).astype(jnp.bfloat16)
    cat_hl = jnp.concatenate([cat_hi, cat_lo], axis=0)
    stats8 = _dotb16(ohT, cat_hl)
    stats = stats8[:, 0:4] + stats8[:, 4:8]
    statsT = stats.T

    counts = stats[:, 3:4]
    centers = stats[:, 0:3] / jnp.maximum(counts, 1.0)
    countsT = statsT[3:4, :]
    centersT = statsT[0:3, :] / jnp.maximum(countsT, 1.0)

    vmT = (countsT >= 2.0).astype(jnp.float32)
    nv = jnp.sum(vmT)
    denom = jnp.maximum(nv, 1.0)

    dx = centers[:, 0:1] - centersT[0:1, :]
    dy = centers[:, 1:2] - centersT[1:2, :]
    dz = centers[:, 2:3] - centersT[2:3, :]
    sq = dx * dx + dy * dy + dz * dz
    pos = sq > 0.0
    dist = jnp.where(pos, jnp.sqrt(jnp.where(pos, sq, 1.0)), 0.0)

    mean_d = jnp.sum(dist * vmT, axis=1, keepdims=True) / denom
    min_d = jnp.min(jnp.where(vmT > 0.0, dist, 1e9), axis=1, keepdims=True)
    zc = centers[:, 2:3]
    frac = jnp.sum((zc > centersT[2:3, :]).astype(jnp.float32) * vmT,
                   axis=1, keepdims=True) / denom
    rel = jnp.concatenate([mean_d, min_d, zc, frac], axis=1)

    h = _gelu(_dot(rel, w1_ref[...]) + b1_ref[...])
    enc = _dot(h, w2_ref[...]) + b2_ref[...]
    a = _dot(enc, wa1_ref[...]) + ba1_ref[...]
    mu = jnp.mean(a, axis=1, keepdims=True)
    var = jnp.mean((a - mu) ** 2, axis=1, keepdims=True)
    a = (a - mu) / jnp.sqrt(var + 1e-5) * lng_ref[...] + lnb_ref[...]
    a = _gelu(a)
    agg = _dot(a, wa2_ref[...]) + ba2_ref[...]

    seg_ok = (counts >= 2.0) & (nv >= 2.0)
    out_ref[pl.ds(b * _K, _K), :] = jnp.where(seg_ok, agg, 0.0)


def _encode(coords_t, labels, W1, b1, W2, b2, Wa1, ba1, ln_g, ln_b,
            Wa2, ba2):
    full = lambda shape: pl.BlockSpec(shape, lambda b: (0,) * len(shape))
    return pl.pallas_call(
        _encode_body,
        grid=(_B // _BPS,),
        in_specs=[
            pl.BlockSpec((_BPS, 3, _N), lambda b: (b, 0, 0)),
            pl.BlockSpec((_BPS, _N), lambda b: (b, 0)),
            full((_D // 4, 4)), full((1, _D // 4)),
            full((_D, _D // 4)), full((1, _D)),
            full((_D, _D)), full((1, _D)),
            full((1, _D)), full((1, _D)),
            full((_D, _D)), full((1, _D)),
        ],
        out_specs=pl.BlockSpec((_BPS * _K, _D), lambda b: (b, 0)),
        out_shape=jax.ShapeDtypeStruct((_B * _K, _D), jnp.float32),
    )(coords_t, labels, W1, b1.reshape(1, -1), W2, b2.reshape(1, -1),
      Wa1, ba1.reshape(1, -1), ln_g.reshape(1, -1), ln_b.reshape(1, -1),
      Wa2, ba2.reshape(1, -1))


_NC, _NS = 2, 16
_NW = _NC * _NS
_RW = (_B * _N) // _NW
_CH = 128
_NCH = _RW // _CH
_NBUF = 3


@functools.cache
def _make_gather_rows():
    @functools.partial(
        pl.kernel,
        out_type=jax.ShapeDtypeStruct((_B * _N, _D), jnp.float32),
        mesh=plsc.VectorSubcoreMesh(core_axis_name="c", subcore_axis_name="s"),
        scratch_types=[
            pltpu.VMEM((_RW,), jnp.int32),
            pltpu.VMEM((_NBUF, _CH, _D), jnp.float32),
        ] + [pltpu.SemaphoreType.DMA] * (2 * _NBUF),
    )
    def _gather_rows(table_hbm, idx_hbm, out_hbm, idx_v, rows_v, *sems):
        gsems, wsems = sems[:_NBUF], sems[_NBUF:]
        wid = lax.axis_index("s") * _NC + lax.axis_index("c")
        wpb = _N // _RW
        batch = wid // wpb
        pltpu.sync_copy(idx_hbm.at[batch, pl.ds((wid % wpb) * _RW, _RW)],
                        idx_v)
        off = batch * _K
        for j in range(_RW // 16):
            sl = pl.ds(j * 16, 16)
            idx_v[sl] = idx_v[sl] + off
        gathers = [None] * _NCH
        writes = [None] * _NCH

        def start_gather(g):
            gathers[g] = pltpu.async_copy(
                table_hbm.at[idx_v.at[pl.ds(g * _CH, _CH)]],
                rows_v.at[g % _NBUF], gsems[g % _NBUF])

        for g in range(_NBUF - 1):
            start_gather(g)
        for g in range(_NCH):
            gathers[g].wait()
            writes[g] = pltpu.async_copy(
                rows_v.at[g % _NBUF],
                out_hbm.at[pl.ds(wid * _RW + g * _CH, _CH)],
                wsems[g % _NBUF])
            nxt = g + _NBUF - 1
            if nxt < _NCH:
                if nxt >= _NBUF:
                    writes[nxt - _NBUF].wait()
                start_gather(nxt)
        for g in range(max(0, _NCH - _NBUF), _NCH):
            writes[g].wait()

    return _gather_rows


def kernel(coordinates, features, superpoint_labels, W1, b1, W2, b2,
           Wa1, ba1, ln_g, ln_b, Wa2, ba2):
    coords_t = coordinates.transpose(0, 2, 1)
    labels = superpoint_labels.astype(jnp.int32)

    table = _encode(coords_t, labels, W1, b1, W2, b2, Wa1, ba1,
                    ln_g, ln_b, Wa2, ba2)

    out = _make_gather_rows()(table, labels)
    return out.reshape(_B, _N, _D).astype(features.dtype)
